# Initial kernel scaffold; baseline (speedup 1.0000x reference)
import jax
import jax.numpy as jnp
from jax.experimental import pallas as pl


def kernel(x, edge_index, edge_attr, batch, params):
    raise NotImplementedError("write your pallas kernel here")


# speedup vs baseline: 1.4590x; 1.4590x over previous
import functools

import jax
import jax.numpy as jnp
from jax import lax
from jax.experimental import pallas as pl
from jax.experimental.pallas import tpu as pltpu
from jax.experimental.pallas import tpu_sc as plsc

_N = 10000
_E = 160000
_F = 128
_D = 16
_G = 64

_NCORE = 2
_NSUB = 16
_NW = _NCORE * _NSUB
_CHUNK = 128
_NCH = 40
_EP = _NW * _NCH * _CHUNK
_NT = 10240
_RPS = _NT // _NSUB

_BE = 2048
_GE = _EP // _BE
_BN = 2000
_GN = _N // _BN

_f32 = jnp.float32


def _mesh():
    return plsc.VectorSubcoreMesh(
        core_axis_name="c", subcore_axis_name="s",
        num_cores=_NCORE, num_subcores=_NSUB)


def _sc_gather_body(src_hbm, x_hbm, out_hbm, idx_v, rows_v, sem):
    cid = lax.axis_index("c")
    sid = lax.axis_index("s")
    wid = sid * _NCORE + cid
    pltpu.sync_copy(src_hbm.at[wid], idx_v)

    @pl.loop(0, _NCH)
    def _chunk(j):
        pltpu.async_copy(x_hbm.at[idx_v.at[j]], rows_v.at[j], sem).wait()

    pltpu.sync_copy(rows_v, out_hbm.at[wid])


def _sc_gather(src_r, x):
    return pl.kernel(
        _sc_gather_body,
        out_type=jax.ShapeDtypeStruct((_NW, _NCH, _CHUNK, _D), _f32),
        mesh=_mesh(),
        compiler_params=pltpu.CompilerParams(use_tc_tiling_on_sc=False),
        scratch_types=[
            pltpu.VMEM((_NCH, _CHUNK), jnp.int32),
            pltpu.VMEM((_NCH, _CHUNK, _D), _f32),
            pltpu.SemaphoreType.DMA,
        ],
    )(src_r, x)


def _sc_scatter_body(dst_hbm, msg_hbm, zeros_hbm, out_hbm, idx_v, msg_v, table):
    cid = lax.axis_index("c")
    sid = lax.axis_index("s")
    wid = sid * _NCORE + cid
    rows = pl.ds(sid * _RPS, _RPS)
    pltpu.sync_copy(zeros_hbm.at[rows], table.at[rows])
    pltpu.sync_copy(dst_hbm.at[wid], idx_v)
    plsc.subcore_barrier()

    @pl.loop(0, _NCH)
    def _chunk(j):
        pltpu.sync_copy(msg_hbm.at[wid, j], msg_v)
        pltpu.sync_copy(msg_v, table.at[idx_v.at[j]], add=True)

    plsc.subcore_barrier()
    pltpu.sync_copy(table.at[rows], out_hbm.at[cid].at[rows])


def _sc_scatter(dst_r, msg_r, zeros, width):
    return pl.kernel(
        _sc_scatter_body,
        out_type=jax.ShapeDtypeStruct((_NCORE, _NT, width), _f32),
        mesh=_mesh(),
        compiler_params=pltpu.CompilerParams(use_tc_tiling_on_sc=False),
        scratch_types=[
            pltpu.VMEM((_NCH, _CHUNK), jnp.int32),
            pltpu.VMEM((_CHUNK, width), _f32),
            pltpu.VMEM_SHARED((_NT, width), _f32),
        ],
    )(dst_r, msg_r, zeros)


def _lin0_body(x_ref, w_ref, b_ref, o_ref):
    y = jnp.dot(x_ref[...], w_ref[...], preferred_element_type=_f32)
    o_ref[...] = jnp.maximum(y + b_ref[...], 0.0)


def _lin0_call(x, w, b):
    return pl.pallas_call(
        _lin0_body,
        grid=(_GN,),
        in_specs=[
            pl.BlockSpec((_BN, _F), lambda i: (i, 0)),
            pl.BlockSpec((_F, _D), lambda i: (0, 0)),
            pl.BlockSpec((1, _D), lambda i: (0, 0)),
        ],
        out_specs=pl.BlockSpec((_BN, _D), lambda i: (i, 0)),
        out_shape=jax.ShapeDtypeStruct((_N, _D), _f32),
    )(x, w, b.reshape(1, _D))


def _edge_msg(ea, xs, w1, b1, w2, b2):
    h = jnp.maximum(jnp.dot(ea, w1, preferred_element_type=_f32) + b1, 0.0)
    w = jnp.dot(h, w2, preferred_element_type=_f32) + b2
    acc = xs[:, 0:1] * w[:, 0:_D]
    for i in range(1, _D):
        acc = acc + xs[:, i:i + 1] * w[:, i * _D:(i + 1) * _D]
    return acc


def _msg_body(ea_ref, xs_ref, w1_ref, b1_ref, w2_ref, b2_ref, o_ref):
    o_ref[...] = _edge_msg(ea_ref[...], xs_ref[...], w1_ref[...], b1_ref[...],
                           w2_ref[...], b2_ref[...])


def _msg_call(ea, xs, w1, b1, w2, b2):
    return pl.pallas_call(
        _msg_body,
        grid=(_GE,),
        in_specs=[
            pl.BlockSpec((_BE, 8), lambda i: (i, 0)),
            pl.BlockSpec((_BE, _D), lambda i: (i, 0)),
            pl.BlockSpec((8, 128), lambda i: (0, 0)),
            pl.BlockSpec((1, 128), lambda i: (0, 0)),
            pl.BlockSpec((128, _D * _D), lambda i: (0, 0)),
            pl.BlockSpec((1, _D * _D), lambda i: (0, 0)),
        ],
        out_specs=pl.BlockSpec((_BE, _D), lambda i: (i, 0)),
        out_shape=jax.ShapeDtypeStruct((_EP, _D), _f32),
    )(ea, xs, w1, b1, w2, b2)


def _msg4_body(ea_ref, xs_ref, w1_ref, b1_ref, w2_ref, b2_ref, o_ref):
    ea = ea_ref[...]
    xs = xs_ref[...]
    for hd in range(4):
        o_ref[:, hd * _D:(hd + 1) * _D] = _edge_msg(
            ea, xs, w1_ref[hd], b1_ref[hd], w2_ref[hd], b2_ref[hd])


def _msg4_call(ea, xs, w1s, b1s, w2s, b2s):
    return pl.pallas_call(
        _msg4_body,
        grid=(_GE,),
        in_specs=[
            pl.BlockSpec((_BE, 8), lambda i: (i, 0)),
            pl.BlockSpec((_BE, _D), lambda i: (i, 0)),
            pl.BlockSpec((4, 8, 128), lambda i: (0, 0, 0)),
            pl.BlockSpec((4, 1, 128), lambda i: (0, 0, 0)),
            pl.BlockSpec((4, 128, _D * _D), lambda i: (0, 0, 0)),
            pl.BlockSpec((4, 1, _D * _D), lambda i: (0, 0, 0)),
        ],
        out_specs=pl.BlockSpec((_BE, 4 * _D), lambda i: (i, 0)),
        out_shape=jax.ShapeDtypeStruct((_EP, 4 * _D), _f32),
    )(ea, xs, w1s, b1s, w2s, b2s)


def _gru_body(aa_ref, ab_ref, ca_ref, cb_ref, bias_ref, wih_ref, whh_ref,
              bih_ref, bhh_ref, h_ref, o_ref):
    cnt = jnp.maximum(ca_ref[...] + cb_ref[...], 1.0)
    m = jnp.maximum((aa_ref[...] + ab_ref[...]) / cnt + bias_ref[...], 0.0)
    h = h_ref[...]
    gi = jnp.dot(m, wih_ref[...], preferred_element_type=_f32) + bih_ref[...]
    gh = jnp.dot(h, whh_ref[...], preferred_element_type=_f32) + bhh_ref[...]
    r = jax.nn.sigmoid(gi[:, 0:_D] + gh[:, 0:_D])
    z = jax.nn.sigmoid(gi[:, _D:2 * _D] + gh[:, _D:2 * _D])
    n = jnp.tanh(gi[:, 2 * _D:3 * _D] + r * gh[:, 2 * _D:3 * _D])
    o_ref[...] = (1.0 - z) * n + z * h


def _gru_call(agg2, cnt2, bias, wih_t, whh_t, bih, bhh, h):
    return pl.pallas_call(
        _gru_body,
        grid=(_GN,),
        in_specs=[
            pl.BlockSpec((_BN, _D), lambda i: (i, 0)),
            pl.BlockSpec((_BN, _D), lambda i: (i, 0)),
            pl.BlockSpec((_BN, _D), lambda i: (i, 0)),
            pl.BlockSpec((_BN, _D), lambda i: (i, 0)),
            pl.BlockSpec((1, _D), lambda i: (0, 0)),
            pl.BlockSpec((_D, 3 * _D), lambda i: (0, 0)),
            pl.BlockSpec((_D, 3 * _D), lambda i: (0, 0)),
            pl.BlockSpec((1, 3 * _D), lambda i: (0, 0)),
            pl.BlockSpec((1, 3 * _D), lambda i: (0, 0)),
            pl.BlockSpec((_BN, _D), lambda i: (i, 0)),
        ],
        out_specs=pl.BlockSpec((_BN, _D), lambda i: (i, 0)),
        out_shape=jax.ShapeDtypeStruct((_N, _D), _f32),
    )(agg2[0], agg2[1], cnt2[0], cnt2[1], bias, wih_t, whh_t, bih, bhh, h)


def _heads_body(aa_ref, ab_ref, ca_ref, cb_ref, bias_ref, o_ref):
    cnt = jnp.maximum(ca_ref[...] + cb_ref[...], 1.0)
    cnt4 = jnp.concatenate([cnt, cnt, cnt, cnt], axis=1)
    o_ref[...] = jnp.maximum(
        (aa_ref[...] + ab_ref[...]) / cnt4 + bias_ref[...], 0.0)


def _heads_call(agg4, cnt2, biases):
    return pl.pallas_call(
        _heads_body,
        grid=(_GN,),
        in_specs=[
            pl.BlockSpec((_BN, 4 * _D), lambda i: (i, 0)),
            pl.BlockSpec((_BN, 4 * _D), lambda i: (i, 0)),
            pl.BlockSpec((_BN, _D), lambda i: (i, 0)),
            pl.BlockSpec((_BN, _D), lambda i: (i, 0)),
            pl.BlockSpec((1, 4 * _D), lambda i: (0, 0)),
        ],
        out_specs=pl.BlockSpec((_BN, 4 * _D), lambda i: (i, 0)),
        out_shape=jax.ShapeDtypeStruct((_N, 4 * _D), _f32),
    )(agg4[0], agg4[1], cnt2[0], cnt2[1], biases)


def _s2s_body(xa_ref, xb_ref, xc_ref, brow_ref, bcol_ref, wih_ref, whh_ref,
              bias_ref, qs_ref, exp_ref):
    iota_gc = lax.broadcasted_iota(jnp.int32, (_G, 1), 0)
    iota_gr = lax.broadcasted_iota(jnp.int32, (1, _G), 1)
    mask_t = brow_ref[...] == iota_gc
    onehot_t = mask_t.astype(_f32)
    onehot = (bcol_ref[...] == iota_gr).astype(_f32)
    xns = (xa_ref, xb_ref, xc_ref)
    for s in range(3):
        xn = xns[s][...]
        wih = wih_ref[s]
        whh = whh_ref[s]
        bias = bias_ref[s]
        q_star = jnp.zeros((2 * _D, _G), _f32)
        h = jnp.zeros((_D, _G), _f32)
        c = jnp.zeros((_D, _G), _f32)
        for _step in range(3):
            gates = (jnp.dot(wih, q_star, preferred_element_type=_f32)
                     + jnp.dot(whh, h, preferred_element_type=_f32) + bias)
            gi = jax.nn.sigmoid(gates[0:_D])
            gf = jax.nn.sigmoid(gates[_D:2 * _D])
            gg = jnp.tanh(gates[2 * _D:3 * _D])
            go = jax.nn.sigmoid(gates[3 * _D:4 * _D])
            c = gf * c + gi * gg
            h = go * jnp.tanh(c)
            q = h
            qb = jnp.dot(q, onehot_t, preferred_element_type=_f32)
            e = jnp.sum(xn * qb, axis=0, keepdims=True)
            masked = jnp.where(mask_t, jnp.broadcast_to(e, (_G, _N)), -1e30)
            emax = jnp.max(masked, axis=1, keepdims=True)
            emax = jnp.where(emax > -1e29, emax, 0.0)
            emax_b = jnp.sum(onehot_t * emax, axis=0, keepdims=True)
            ee = jnp.exp(e - emax_b)
            denom = jnp.sum(onehot_t * ee, axis=1, keepdims=True)
            denom_b = jnp.sum(onehot_t * denom, axis=0, keepdims=True)
            a = ee / (denom_b + 1e-16)
            r = jnp.dot(xn * a, onehot, preferred_element_type=_f32)
            q_star = jnp.concatenate([q, r], axis=0)
        qs_ref[s] = q_star
        if s >= 1:
            exp_ref[s - 1] = jnp.dot(q_star, onehot_t,
                                     preferred_element_type=_f32)


def _s2s_call(xa_t, xb_t, xc_t, brow, bcol, wih_s, whh_s, bias_s):
    return pl.pallas_call(
        _s2s_body,
        out_shape=(
            jax.ShapeDtypeStruct((3, 2 * _D, _G), _f32),
            jax.ShapeDtypeStruct((2, 2 * _D, _N), _f32),
        ),
    )(xa_t, xb_t, xc_t, brow, bcol, wih_s, whh_s, bias_s)


def kernel(x, edge_index, edge_attr, batch, params):
    p = params
    src = edge_index[0]
    dst = edge_index[1]
    ea = jnp.pad(edge_attr, ((0, _EP - _E), (0, 3)))
    src_r = jnp.pad(src, (0, _EP - _E)).reshape(_NW, _NCH, _CHUNK)
    dst_r = jnp.pad(dst, (0, _EP - _E),
                    constant_values=_N).reshape(_NW, _NCH, _CHUNK)
    zeros16 = jnp.zeros((_NT, _D), _f32)
    zeros64 = jnp.zeros((_NT, 4 * _D), _f32)
    ones_msg = jnp.ones((_NW, _NCH, _CHUNK, _D), _f32)

    cnt2 = _sc_scatter(dst_r, ones_msg, zeros16, _D)

    cw1 = jnp.pad(p['conv_W1'], ((0, 3), (0, 0)))
    cb1 = p['conv_b1'].reshape(1, 128)
    cw2 = p['conv_W2']
    cb2 = p['conv_b2'].reshape(1, _D * _D)
    cbias = p['conv_bias'].reshape(1, _D)
    wih_t = p['gru_Wih'].T
    whh_t = p['gru_Whh'].T
    bih = p['gru_bih'].reshape(1, 3 * _D)
    bhh = p['gru_bhh'].reshape(1, 3 * _D)

    out = _lin0_call(x, p['lin0_W'], p['lin0_b'])
    h = out
    for _ in range(3):
        xs = _sc_gather(src_r, out).reshape(_EP, _D)
        msg = _msg_call(ea, xs, cw1, cb1, cw2, cb2)
        agg2 = _sc_scatter(dst_r, msg.reshape(_NW, _NCH, _CHUNK, _D),
                           zeros16, _D)
        h = _gru_call(agg2, cnt2, cbias, wih_t, whh_t, bih, bhh, h)
        out = h

    heads = ['node_mu', 'node_lv', 'graph_mu', 'graph_lv']
    w1s = jnp.stack([jnp.pad(p[n + '_W1'], ((0, 3), (0, 0))) for n in heads])
    b1s = jnp.stack([p[n + '_b1'].reshape(1, 128) for n in heads])
    w2s = jnp.stack([p[n + '_W2'] for n in heads])
    b2s = jnp.stack([p[n + '_b2'].reshape(1, _D * _D) for n in heads])
    biases4 = jnp.concatenate([p[n + '_bias'] for n in heads]).reshape(1, 4 * _D)

    xs = _sc_gather(src_r, out).reshape(_EP, _D)
    msg4 = _msg4_call(ea, xs, w1s, b1s, w2s, b2s)
    agg4 = _sc_scatter(dst_r, msg4.reshape(_NW, _NCH, _CHUNK, 4 * _D),
                       zeros64, 4 * _D)
    hout = _heads_call(agg4, cnt2, biases4)

    node_mu = hout[:, 0:_D]
    node_lv = hout[:, _D:2 * _D]
    gmu_id = hout[:, 2 * _D:3 * _D]
    glv_id = hout[:, 3 * _D:4 * _D]

    s2s = ['s2s_nodes', 's2s_mu', 's2s_lv']
    wih_s = jnp.stack([p[n + '_Wih'] for n in s2s])
    whh_s = jnp.stack([p[n + '_Whh'] for n in s2s])
    bias_s = jnp.stack([(p[n + '_bih'] + p[n + '_bhh']).reshape(4 * _D, 1)
                        for n in s2s])
    brow = batch.reshape(1, _N)
    bcol = batch.reshape(_N, 1)
    qs, exp = _s2s_call(out.T, gmu_id.T, glv_id.T, brow, bcol,
                        wih_s, whh_s, bias_s)

    node_graph = qs[0].T
    grouped_mu_expanded = exp[0].T
    grouped_lv_expanded = exp[1].T
    return (node_mu, node_lv, grouped_mu_expanded, grouped_lv_expanded,
            node_graph)


# speedup vs baseline: 4.1923x; 2.8734x over previous
import functools

import jax
import jax.numpy as jnp
from jax import lax
from jax.experimental import pallas as pl
from jax.experimental.pallas import tpu as pltpu
from jax.experimental.pallas import tpu_sc as plsc

_N = 10000
_E = 160000
_F = 128
_D = 16
_G = 64

_NCORE = 2
_NSUB = 16
_NW = _NCORE * _NSUB
_CHUNK = 128
_NCH = 40
_EP = _NW * _NCH * _CHUNK
_NT = 10240
_RPS = _NT // _NSUB

_BE = 2048
_GE = _EP // _BE
_BN = 2000
_GN = _N // _BN

_f32 = jnp.float32


def _mesh():
    return plsc.VectorSubcoreMesh(
        core_axis_name="c", subcore_axis_name="s",
        num_cores=_NCORE, num_subcores=_NSUB)


def _sc_gather_body(src_hbm, x_hbm, out_hbm, idx_v, rows_v, sem):
    cid = lax.axis_index("c")
    sid = lax.axis_index("s")
    wid = sid * _NCORE + cid
    pltpu.sync_copy(src_hbm.at[wid], idx_v)

    @pl.loop(0, _NCH)
    def _chunk(j):
        pltpu.async_copy(x_hbm.at[idx_v.at[j]], rows_v.at[j], sem).wait()

    pltpu.sync_copy(rows_v, out_hbm.at[wid])


def _sc_gather(src_r, x):
    return pl.kernel(
        _sc_gather_body,
        out_type=jax.ShapeDtypeStruct((_NW, _NCH, _CHUNK, _D), _f32),
        mesh=_mesh(),
        compiler_params=pltpu.CompilerParams(use_tc_tiling_on_sc=False),
        scratch_types=[
            pltpu.VMEM((_NCH, _CHUNK), jnp.int32),
            pltpu.VMEM((_NCH, _CHUNK, _D), _f32),
            pltpu.SemaphoreType.DMA,
        ],
    )(src_r, x)


def _sc_scatter_body(dst_hbm, msg_hbm, zeros_hbm, out_hbm, idx_v, msg_v, table):
    cid = lax.axis_index("c")
    sid = lax.axis_index("s")
    wid = sid * _NCORE + cid
    rows = pl.ds(sid * _RPS, _RPS)
    pltpu.sync_copy(zeros_hbm.at[rows], table.at[rows])
    pltpu.sync_copy(dst_hbm.at[wid], idx_v)
    plsc.subcore_barrier()

    @pl.loop(0, _NCH)
    def _chunk(j):
        pltpu.sync_copy(msg_hbm.at[wid, j], msg_v)
        pltpu.sync_copy(msg_v, table.at[idx_v.at[j]], add=True)

    plsc.subcore_barrier()
    pltpu.sync_copy(table.at[rows], out_hbm.at[cid].at[rows])


def _sc_scatter(dst_r, msg_r, zeros, width):
    return pl.kernel(
        _sc_scatter_body,
        out_type=jax.ShapeDtypeStruct((_NCORE, _NT, width), _f32),
        mesh=_mesh(),
        compiler_params=pltpu.CompilerParams(use_tc_tiling_on_sc=False),
        scratch_types=[
            pltpu.VMEM((_NCH, _CHUNK), jnp.int32),
            pltpu.VMEM((_CHUNK, width), _f32),
            pltpu.VMEM_SHARED((_NT, width), _f32),
        ],
    )(dst_r, msg_r, zeros)


def _lin0_body(x_ref, w_ref, b_ref, o_ref):
    y = jnp.dot(x_ref[...], w_ref[...], preferred_element_type=_f32)
    o_ref[...] = jnp.maximum(y + b_ref[...], 0.0)


def _lin0_call(x, w, b):
    return pl.pallas_call(
        _lin0_body,
        grid=(_GN,),
        in_specs=[
            pl.BlockSpec((_BN, _F), lambda i: (i, 0)),
            pl.BlockSpec((_F, _D), lambda i: (0, 0)),
            pl.BlockSpec((1, _D), lambda i: (0, 0)),
        ],
        out_specs=pl.BlockSpec((_BN, _D), lambda i: (i, 0)),
        out_shape=jax.ShapeDtypeStruct((_N, _D), _f32),
    )(x, w, b.reshape(1, _D))


def _spread_fold():
    ri = lax.broadcasted_iota(jnp.int32, (_D, _D * _D), 0)
    rc = lax.broadcasted_iota(jnp.int32, (_D, _D * _D), 1)
    spread = (ri == rc // _D).astype(_f32)
    si = lax.broadcasted_iota(jnp.int32, (_D * _D, _D), 0)
    sc = lax.broadcasted_iota(jnp.int32, (_D * _D, _D), 1)
    fold = (si % _D == sc).astype(_f32)
    return spread, fold


def _edge_msg(ea, xs, w1, b1, w2, b2, spread, fold):
    h = jnp.maximum(jnp.dot(ea, w1, preferred_element_type=_f32) + b1, 0.0)
    w = jnp.dot(h, w2, preferred_element_type=_f32) + b2
    xr = jnp.dot(xs, spread, preferred_element_type=_f32)
    return jnp.dot(xr * w, fold, preferred_element_type=_f32)


def _msg_body(ea_ref, xs_ref, w1_ref, b1_ref, w2_ref, b2_ref, o_ref):
    spread, fold = _spread_fold()
    o_ref[...] = _edge_msg(ea_ref[...], xs_ref[...], w1_ref[...], b1_ref[...],
                           w2_ref[...], b2_ref[...], spread, fold)


def _msg_call(ea, xs, w1, b1, w2, b2):
    return pl.pallas_call(
        _msg_body,
        grid=(_GE,),
        in_specs=[
            pl.BlockSpec((_BE, 8), lambda i: (i, 0)),
            pl.BlockSpec((_BE, _D), lambda i: (i, 0)),
            pl.BlockSpec((8, 128), lambda i: (0, 0)),
            pl.BlockSpec((1, 128), lambda i: (0, 0)),
            pl.BlockSpec((128, _D * _D), lambda i: (0, 0)),
            pl.BlockSpec((1, _D * _D), lambda i: (0, 0)),
        ],
        out_specs=pl.BlockSpec((_BE, _D), lambda i: (i, 0)),
        out_shape=jax.ShapeDtypeStruct((_EP, _D), _f32),
    )(ea, xs, w1, b1, w2, b2)


def _msg4_body(ea_ref, xs_ref, w1_ref, b1_ref, w2_ref, b2_ref, o_ref):
    ea = ea_ref[...]
    xs = xs_ref[...]
    spread, fold = _spread_fold()
    for hd in range(4):
        o_ref[:, hd * _D:(hd + 1) * _D] = _edge_msg(
            ea, xs, w1_ref[hd], b1_ref[hd], w2_ref[hd], b2_ref[hd],
            spread, fold)


def _msg4_call(ea, xs, w1s, b1s, w2s, b2s):
    return pl.pallas_call(
        _msg4_body,
        grid=(_GE,),
        in_specs=[
            pl.BlockSpec((_BE, 8), lambda i: (i, 0)),
            pl.BlockSpec((_BE, _D), lambda i: (i, 0)),
            pl.BlockSpec((4, 8, 128), lambda i: (0, 0, 0)),
            pl.BlockSpec((4, 1, 128), lambda i: (0, 0, 0)),
            pl.BlockSpec((4, 128, _D * _D), lambda i: (0, 0, 0)),
            pl.BlockSpec((4, 1, _D * _D), lambda i: (0, 0, 0)),
        ],
        out_specs=pl.BlockSpec((_BE, 4 * _D), lambda i: (i, 0)),
        out_shape=jax.ShapeDtypeStruct((_EP, 4 * _D), _f32),
    )(ea, xs, w1s, b1s, w2s, b2s)


def _gru_body(aa_ref, ab_ref, ca_ref, cb_ref, bias_ref, wih_ref, whh_ref,
              bih_ref, bhh_ref, h_ref, o_ref):
    cnt = jnp.maximum(ca_ref[...] + cb_ref[...], 1.0)
    m = jnp.maximum((aa_ref[...] + ab_ref[...]) / cnt + bias_ref[...], 0.0)
    h = h_ref[...]
    gi = jnp.dot(m, wih_ref[...], preferred_element_type=_f32) + bih_ref[...]
    gh = jnp.dot(h, whh_ref[...], preferred_element_type=_f32) + bhh_ref[...]
    r = jax.nn.sigmoid(gi[:, 0:_D] + gh[:, 0:_D])
    z = jax.nn.sigmoid(gi[:, _D:2 * _D] + gh[:, _D:2 * _D])
    n = jnp.tanh(gi[:, 2 * _D:3 * _D] + r * gh[:, 2 * _D:3 * _D])
    o_ref[...] = (1.0 - z) * n + z * h


def _gru_call(agg2, cnt2, bias, wih_t, whh_t, bih, bhh, h):
    return pl.pallas_call(
        _gru_body,
        grid=(_GN,),
        in_specs=[
            pl.BlockSpec((_BN, _D), lambda i: (i, 0)),
            pl.BlockSpec((_BN, _D), lambda i: (i, 0)),
            pl.BlockSpec((_BN, _D), lambda i: (i, 0)),
            pl.BlockSpec((_BN, _D), lambda i: (i, 0)),
            pl.BlockSpec((1, _D), lambda i: (0, 0)),
            pl.BlockSpec((_D, 3 * _D), lambda i: (0, 0)),
            pl.BlockSpec((_D, 3 * _D), lambda i: (0, 0)),
            pl.BlockSpec((1, 3 * _D), lambda i: (0, 0)),
            pl.BlockSpec((1, 3 * _D), lambda i: (0, 0)),
            pl.BlockSpec((_BN, _D), lambda i: (i, 0)),
        ],
        out_specs=pl.BlockSpec((_BN, _D), lambda i: (i, 0)),
        out_shape=jax.ShapeDtypeStruct((_N, _D), _f32),
    )(agg2[0], agg2[1], cnt2[0], cnt2[1], bias, wih_t, whh_t, bih, bhh, h)


def _heads_body(aa_ref, ab_ref, ca_ref, cb_ref, bias_ref, o_ref):
    cnt = jnp.maximum(ca_ref[...] + cb_ref[...], 1.0)
    cnt4 = jnp.concatenate([cnt, cnt, cnt, cnt], axis=1)
    o_ref[...] = jnp.maximum(
        (aa_ref[...] + ab_ref[...]) / cnt4 + bias_ref[...], 0.0)


def _heads_call(agg4, cnt2, biases):
    return pl.pallas_call(
        _heads_body,
        grid=(_GN,),
        in_specs=[
            pl.BlockSpec((_BN, 4 * _D), lambda i: (i, 0)),
            pl.BlockSpec((_BN, 4 * _D), lambda i: (i, 0)),
            pl.BlockSpec((_BN, _D), lambda i: (i, 0)),
            pl.BlockSpec((_BN, _D), lambda i: (i, 0)),
            pl.BlockSpec((1, 4 * _D), lambda i: (0, 0)),
        ],
        out_specs=pl.BlockSpec((_BN, 4 * _D), lambda i: (i, 0)),
        out_shape=jax.ShapeDtypeStruct((_N, 4 * _D), _f32),
    )(agg4[0], agg4[1], cnt2[0], cnt2[1], biases)


def _s2s_body(xa_ref, xb_ref, xc_ref, brow_ref, bcol_ref, wih_ref, whh_ref,
              bias_ref, qs_ref, exp_ref):
    iota_gc = lax.broadcasted_iota(jnp.int32, (_G, 1), 0)
    iota_gr = lax.broadcasted_iota(jnp.int32, (1, _G), 1)
    mask_t = brow_ref[...] == iota_gc
    onehot_t = mask_t.astype(_f32)
    onehot = (bcol_ref[...] == iota_gr).astype(_f32)
    xns = (xa_ref, xb_ref, xc_ref)
    for s in range(3):
        xn = xns[s][...]
        wih = wih_ref[s]
        whh = whh_ref[s]
        bias = bias_ref[s]
        q_star = jnp.zeros((2 * _D, _G), _f32)
        h = jnp.zeros((_D, _G), _f32)
        c = jnp.zeros((_D, _G), _f32)
        for _step in range(3):
            gates = (jnp.dot(wih, q_star, preferred_element_type=_f32)
                     + jnp.dot(whh, h, preferred_element_type=_f32) + bias)
            gi = jax.nn.sigmoid(gates[0:_D])
            gf = jax.nn.sigmoid(gates[_D:2 * _D])
            gg = jnp.tanh(gates[2 * _D:3 * _D])
            go = jax.nn.sigmoid(gates[3 * _D:4 * _D])
            c = gf * c + gi * gg
            h = go * jnp.tanh(c)
            q = h
            qb = jnp.dot(q, onehot_t, preferred_element_type=_f32)
            e = jnp.sum(xn * qb, axis=0, keepdims=True)
            masked = jnp.where(mask_t, jnp.broadcast_to(e, (_G, _N)), -1e30)
            emax = jnp.max(masked, axis=1, keepdims=True)
            emax = jnp.where(emax > -1e29, emax, 0.0)
            emax_b = jnp.sum(onehot_t * emax, axis=0, keepdims=True)
            ee = jnp.exp(e - emax_b)
            denom = jnp.sum(onehot_t * ee, axis=1, keepdims=True)
            denom_b = jnp.sum(onehot_t * denom, axis=0, keepdims=True)
            a = ee / (denom_b + 1e-16)
            r = jnp.dot(xn * a, onehot, preferred_element_type=_f32)
            q_star = jnp.concatenate([q, r], axis=0)
        qs_ref[s] = q_star
        if s >= 1:
            exp_ref[s - 1] = jnp.dot(q_star, onehot_t,
                                     preferred_element_type=_f32)


def _s2s_call(xa_t, xb_t, xc_t, brow, bcol, wih_s, whh_s, bias_s):
    return pl.pallas_call(
        _s2s_body,
        out_shape=(
            jax.ShapeDtypeStruct((3, 2 * _D, _G), _f32),
            jax.ShapeDtypeStruct((2, 2 * _D, _N), _f32),
        ),
    )(xa_t, xb_t, xc_t, brow, bcol, wih_s, whh_s, bias_s)


def kernel(x, edge_index, edge_attr, batch, params):
    p = params
    src = edge_index[0]
    dst = edge_index[1]
    ea = jnp.pad(edge_attr, ((0, _EP - _E), (0, 3)))
    src_r = jnp.pad(src, (0, _EP - _E)).reshape(_NW, _NCH, _CHUNK)
    dst_r = jnp.pad(dst, (0, _EP - _E),
                    constant_values=_N).reshape(_NW, _NCH, _CHUNK)
    zeros16 = jnp.zeros((_NT, _D), _f32)
    zeros64 = jnp.zeros((_NT, 4 * _D), _f32)
    ones_msg = jnp.ones((_NW, _NCH, _CHUNK, _D), _f32)

    cnt2 = _sc_scatter(dst_r, ones_msg, zeros16, _D)

    cw1 = jnp.pad(p['conv_W1'], ((0, 3), (0, 0)))
    cb1 = p['conv_b1'].reshape(1, 128)
    cw2 = p['conv_W2']
    cb2 = p['conv_b2'].reshape(1, _D * _D)
    cbias = p['conv_bias'].reshape(1, _D)
    wih_t = p['gru_Wih'].T
    whh_t = p['gru_Whh'].T
    bih = p['gru_bih'].reshape(1, 3 * _D)
    bhh = p['gru_bhh'].reshape(1, 3 * _D)

    out = _lin0_call(x, p['lin0_W'], p['lin0_b'])
    h = out
    for _ in range(3):
        xs = _sc_gather(src_r, out).reshape(_EP, _D)
        msg = _msg_call(ea, xs, cw1, cb1, cw2, cb2)
        agg2 = _sc_scatter(dst_r, msg.reshape(_NW, _NCH, _CHUNK, _D),
                           zeros16, _D)
        h = _gru_call(agg2, cnt2, cbias, wih_t, whh_t, bih, bhh, h)
        out = h

    heads = ['node_mu', 'node_lv', 'graph_mu', 'graph_lv']
    w1s = jnp.stack([jnp.pad(p[n + '_W1'], ((0, 3), (0, 0))) for n in heads])
    b1s = jnp.stack([p[n + '_b1'].reshape(1, 128) for n in heads])
    w2s = jnp.stack([p[n + '_W2'] for n in heads])
    b2s = jnp.stack([p[n + '_b2'].reshape(1, _D * _D) for n in heads])
    biases4 = jnp.concatenate([p[n + '_bias'] for n in heads]).reshape(1, 4 * _D)

    xs = _sc_gather(src_r, out).reshape(_EP, _D)
    msg4 = _msg4_call(ea, xs, w1s, b1s, w2s, b2s)
    agg4 = _sc_scatter(dst_r, msg4.reshape(_NW, _NCH, _CHUNK, 4 * _D),
                       zeros64, 4 * _D)
    hout = _heads_call(agg4, cnt2, biases4)

    node_mu = hout[:, 0:_D]
    node_lv = hout[:, _D:2 * _D]
    gmu_id = hout[:, 2 * _D:3 * _D]
    glv_id = hout[:, 3 * _D:4 * _D]

    s2s = ['s2s_nodes', 's2s_mu', 's2s_lv']
    wih_s = jnp.stack([p[n + '_Wih'] for n in s2s])
    whh_s = jnp.stack([p[n + '_Whh'] for n in s2s])
    bias_s = jnp.stack([(p[n + '_bih'] + p[n + '_bhh']).reshape(4 * _D, 1)
                        for n in s2s])
    brow = batch.reshape(1, _N)
    bcol = batch.reshape(_N, 1)
    qs, exp = _s2s_call(out.T, gmu_id.T, glv_id.T, brow, bcol,
                        wih_s, whh_s, bias_s)

    node_graph = qs[0].T
    grouped_mu_expanded = exp[0].T
    grouped_lv_expanded = exp[1].T
    return (node_mu, node_lv, grouped_mu_expanded, grouped_lv_expanded,
            node_graph)


# speedup vs baseline: 4.3524x; 1.0382x over previous
import functools

import jax
import jax.numpy as jnp
from jax import lax
from jax.experimental import pallas as pl
from jax.experimental.pallas import tpu as pltpu
from jax.experimental.pallas import tpu_sc as plsc

_N = 10000
_E = 160000
_F = 128
_D = 16
_G = 64

_NCORE = 2
_NSUB = 16
_NW = _NCORE * _NSUB
_CHUNK = 128
_NCH = 40
_EP = _NW * _NCH * _CHUNK
_NT = 10240
_RPS = _NT // _NSUB

_BE = 2048
_GE = _EP // _BE
_BN = 2000
_GN = _N // _BN

_f32 = jnp.float32


def _mesh():
    return plsc.VectorSubcoreMesh(
        core_axis_name="c", subcore_axis_name="s",
        num_cores=_NCORE, num_subcores=_NSUB)


def _sc_gather_body(src_hbm, x_hbm, out_hbm, idx_v, rows_v, sem):
    cid = lax.axis_index("c")
    sid = lax.axis_index("s")
    wid = sid * _NCORE + cid
    pltpu.sync_copy(src_hbm.at[wid], idx_v)

    @pl.loop(0, _NCH, step=8)
    def _grp(j0):
        descs = [
            pltpu.async_copy(x_hbm.at[idx_v.at[j0 + b]], rows_v.at[j0 + b],
                             sem)
            for b in range(8)
        ]
        for d in descs:
            d.wait()

    pltpu.sync_copy(rows_v, out_hbm.at[wid])


def _sc_gather(src_r, x):
    return pl.kernel(
        _sc_gather_body,
        out_type=jax.ShapeDtypeStruct((_NW, _NCH, _CHUNK, _D), _f32),
        mesh=_mesh(),
        compiler_params=pltpu.CompilerParams(use_tc_tiling_on_sc=False),
        scratch_types=[
            pltpu.VMEM((_NCH, _CHUNK), jnp.int32),
            pltpu.VMEM((_NCH, _CHUNK, _D), _f32),
            pltpu.SemaphoreType.DMA,
        ],
    )(src_r, x)


def _sc_scatter_body(dst_hbm, msg_hbm, zeros_hbm, out_hbm, idx_v, msg_v, table,
                     lsem):
    cid = lax.axis_index("c")
    sid = lax.axis_index("s")
    wid = sid * _NCORE + cid
    rows = pl.ds(sid * _RPS, _RPS)
    pltpu.sync_copy(zeros_hbm.at[rows], table.at[rows])
    pltpu.sync_copy(dst_hbm.at[wid], idx_v)
    plsc.subcore_barrier()
    pltpu.async_copy(msg_hbm.at[wid, 0], msg_v.at[0], lsem)

    @pl.loop(0, _NCH)
    def _chunk(j):
        b = lax.rem(j, 2)
        pltpu.make_async_copy(msg_hbm.at[wid, j], msg_v.at[b], lsem).wait()

        @pl.when(j < _NCH - 1)
        def _prefetch():
            pltpu.async_copy(msg_hbm.at[wid, j + 1], msg_v.at[1 - b], lsem)

        pltpu.sync_copy(msg_v.at[b], table.at[idx_v.at[j]], add=True)

    plsc.subcore_barrier()
    pltpu.sync_copy(table.at[rows], out_hbm.at[cid].at[rows])


def _sc_scatter(dst_r, msg_r, zeros, width):
    return pl.kernel(
        _sc_scatter_body,
        out_type=jax.ShapeDtypeStruct((_NCORE, _NT, width), _f32),
        mesh=_mesh(),
        compiler_params=pltpu.CompilerParams(use_tc_tiling_on_sc=False),
        scratch_types=[
            pltpu.VMEM((_NCH, _CHUNK), jnp.int32),
            pltpu.VMEM((2, _CHUNK, width), _f32),
            pltpu.VMEM_SHARED((_NT, width), _f32),
            pltpu.SemaphoreType.DMA,
        ],
    )(dst_r, msg_r, zeros)


def _lin0_body(x_ref, w_ref, b_ref, o_ref):
    y = jnp.dot(x_ref[...], w_ref[...], preferred_element_type=_f32)
    o_ref[...] = jnp.maximum(y + b_ref[...], 0.0)


def _lin0_call(x, w, b):
    return pl.pallas_call(
        _lin0_body,
        grid=(_GN,),
        in_specs=[
            pl.BlockSpec((_BN, _F), lambda i: (i, 0)),
            pl.BlockSpec((_F, _D), lambda i: (0, 0)),
            pl.BlockSpec((1, _D), lambda i: (0, 0)),
        ],
        out_specs=pl.BlockSpec((_BN, _D), lambda i: (i, 0)),
        out_shape=jax.ShapeDtypeStruct((_N, _D), _f32),
    )(x, w, b.reshape(1, _D))


def _spread_fold():
    ri = lax.broadcasted_iota(jnp.int32, (_D, _D * _D), 0)
    rc = lax.broadcasted_iota(jnp.int32, (_D, _D * _D), 1)
    spread = (ri == rc // _D).astype(_f32)
    si = lax.broadcasted_iota(jnp.int32, (_D * _D, _D), 0)
    sc = lax.broadcasted_iota(jnp.int32, (_D * _D, _D), 1)
    fold = (si % _D == sc).astype(_f32)
    return spread, fold


def _edge_msg(ea, xs, w1, b1, w2, b2, spread, fold):
    h = jnp.maximum(jnp.dot(ea, w1, preferred_element_type=_f32) + b1, 0.0)
    w = jnp.dot(h, w2, preferred_element_type=_f32) + b2
    xr = jnp.dot(xs, spread, preferred_element_type=_f32)
    return jnp.dot(xr * w, fold, preferred_element_type=_f32)


def _msg_body(ea_ref, xs_ref, w1_ref, b1_ref, w2_ref, b2_ref, o_ref):
    spread, fold = _spread_fold()
    o_ref[...] = _edge_msg(ea_ref[...], xs_ref[...], w1_ref[...], b1_ref[...],
                           w2_ref[...], b2_ref[...], spread, fold)


def _msg_call(ea, xs, w1, b1, w2, b2):
    return pl.pallas_call(
        _msg_body,
        grid=(_GE,),
        in_specs=[
            pl.BlockSpec((_BE, 8), lambda i: (i, 0)),
            pl.BlockSpec((_BE, _D), lambda i: (i, 0)),
            pl.BlockSpec((8, 128), lambda i: (0, 0)),
            pl.BlockSpec((1, 128), lambda i: (0, 0)),
            pl.BlockSpec((128, _D * _D), lambda i: (0, 0)),
            pl.BlockSpec((1, _D * _D), lambda i: (0, 0)),
        ],
        out_specs=pl.BlockSpec((_BE, _D), lambda i: (i, 0)),
        out_shape=jax.ShapeDtypeStruct((_EP, _D), _f32),
    )(ea, xs, w1, b1, w2, b2)


def _msg4_body(ea_ref, xs_ref, w1_ref, b1_ref, w2_ref, b2_ref, o_ref):
    ea = ea_ref[...]
    xs = xs_ref[...]
    spread, fold = _spread_fold()
    for hd in range(4):
        o_ref[:, hd * _D:(hd + 1) * _D] = _edge_msg(
            ea, xs, w1_ref[hd], b1_ref[hd], w2_ref[hd], b2_ref[hd],
            spread, fold)


def _msg4_call(ea, xs, w1s, b1s, w2s, b2s):
    return pl.pallas_call(
        _msg4_body,
        grid=(_GE,),
        in_specs=[
            pl.BlockSpec((_BE, 8), lambda i: (i, 0)),
            pl.BlockSpec((_BE, _D), lambda i: (i, 0)),
            pl.BlockSpec((4, 8, 128), lambda i: (0, 0, 0)),
            pl.BlockSpec((4, 1, 128), lambda i: (0, 0, 0)),
            pl.BlockSpec((4, 128, _D * _D), lambda i: (0, 0, 0)),
            pl.BlockSpec((4, 1, _D * _D), lambda i: (0, 0, 0)),
        ],
        out_specs=pl.BlockSpec((_BE, 4 * _D), lambda i: (i, 0)),
        out_shape=jax.ShapeDtypeStruct((_EP, 4 * _D), _f32),
    )(ea, xs, w1s, b1s, w2s, b2s)


def _gru_body(aa_ref, ab_ref, ca_ref, cb_ref, bias_ref, wih_ref, whh_ref,
              bih_ref, bhh_ref, h_ref, o_ref):
    cnt = jnp.maximum(ca_ref[...] + cb_ref[...], 1.0)
    m = jnp.maximum((aa_ref[...] + ab_ref[...]) / cnt + bias_ref[...], 0.0)
    h = h_ref[...]
    gi = jnp.dot(m, wih_ref[...], preferred_element_type=_f32) + bih_ref[...]
    gh = jnp.dot(h, whh_ref[...], preferred_element_type=_f32) + bhh_ref[...]
    r = jax.nn.sigmoid(gi[:, 0:_D] + gh[:, 0:_D])
    z = jax.nn.sigmoid(gi[:, _D:2 * _D] + gh[:, _D:2 * _D])
    n = jnp.tanh(gi[:, 2 * _D:3 * _D] + r * gh[:, 2 * _D:3 * _D])
    o_ref[...] = (1.0 - z) * n + z * h


def _gru_call(agg2, cnt2, bias, wih_t, whh_t, bih, bhh, h):
    return pl.pallas_call(
        _gru_body,
        grid=(_GN,),
        in_specs=[
            pl.BlockSpec((_BN, _D), lambda i: (i, 0)),
            pl.BlockSpec((_BN, _D), lambda i: (i, 0)),
            pl.BlockSpec((_BN, _D), lambda i: (i, 0)),
            pl.BlockSpec((_BN, _D), lambda i: (i, 0)),
            pl.BlockSpec((1, _D), lambda i: (0, 0)),
            pl.BlockSpec((_D, 3 * _D), lambda i: (0, 0)),
            pl.BlockSpec((_D, 3 * _D), lambda i: (0, 0)),
            pl.BlockSpec((1, 3 * _D), lambda i: (0, 0)),
            pl.BlockSpec((1, 3 * _D), lambda i: (0, 0)),
            pl.BlockSpec((_BN, _D), lambda i: (i, 0)),
        ],
        out_specs=pl.BlockSpec((_BN, _D), lambda i: (i, 0)),
        out_shape=jax.ShapeDtypeStruct((_N, _D), _f32),
    )(agg2[0], agg2[1], cnt2[0], cnt2[1], bias, wih_t, whh_t, bih, bhh, h)


def _heads_body(aa_ref, ab_ref, ca_ref, cb_ref, bias_ref, o_ref):
    cnt = jnp.maximum(ca_ref[...] + cb_ref[...], 1.0)
    cnt4 = jnp.concatenate([cnt, cnt, cnt, cnt], axis=1)
    o_ref[...] = jnp.maximum(
        (aa_ref[...] + ab_ref[...]) / cnt4 + bias_ref[...], 0.0)


def _heads_call(agg4, cnt2, biases):
    return pl.pallas_call(
        _heads_body,
        grid=(_GN,),
        in_specs=[
            pl.BlockSpec((_BN, 4 * _D), lambda i: (i, 0)),
            pl.BlockSpec((_BN, 4 * _D), lambda i: (i, 0)),
            pl.BlockSpec((_BN, _D), lambda i: (i, 0)),
            pl.BlockSpec((_BN, _D), lambda i: (i, 0)),
            pl.BlockSpec((1, 4 * _D), lambda i: (0, 0)),
        ],
        out_specs=pl.BlockSpec((_BN, 4 * _D), lambda i: (i, 0)),
        out_shape=jax.ShapeDtypeStruct((_N, 4 * _D), _f32),
    )(agg4[0], agg4[1], cnt2[0], cnt2[1], biases)


def _s2s_body(xa_ref, xb_ref, xc_ref, brow_ref, bcol_ref, wih_ref, whh_ref,
              bias_ref, qs_ref, exp_ref):
    iota_gc = lax.broadcasted_iota(jnp.int32, (_G, 1), 0)
    iota_gr = lax.broadcasted_iota(jnp.int32, (1, _G), 1)
    mask_t = brow_ref[...] == iota_gc
    onehot_t = mask_t.astype(_f32)
    onehot = (bcol_ref[...] == iota_gr).astype(_f32)
    xns = (xa_ref, xb_ref, xc_ref)
    for s in range(3):
        xn = xns[s][...]
        wih = wih_ref[s]
        whh = whh_ref[s]
        bias = bias_ref[s]
        q_star = jnp.zeros((2 * _D, _G), _f32)
        h = jnp.zeros((_D, _G), _f32)
        c = jnp.zeros((_D, _G), _f32)
        for _step in range(3):
            gates = (jnp.dot(wih, q_star, preferred_element_type=_f32)
                     + jnp.dot(whh, h, preferred_element_type=_f32) + bias)
            gi = jax.nn.sigmoid(gates[0:_D])
            gf = jax.nn.sigmoid(gates[_D:2 * _D])
            gg = jnp.tanh(gates[2 * _D:3 * _D])
            go = jax.nn.sigmoid(gates[3 * _D:4 * _D])
            c = gf * c + gi * gg
            h = go * jnp.tanh(c)
            q = h
            qb = jnp.dot(q, onehot_t, preferred_element_type=_f32)
            e = jnp.sum(xn * qb, axis=0, keepdims=True)
            masked = jnp.where(mask_t, jnp.broadcast_to(e, (_G, _N)), -1e30)
            emax = jnp.max(masked, axis=1, keepdims=True)
            emax = jnp.where(emax > -1e29, emax, 0.0)
            emax_b = jnp.sum(onehot_t * emax, axis=0, keepdims=True)
            ee = jnp.exp(e - emax_b)
            denom = jnp.sum(onehot_t * ee, axis=1, keepdims=True)
            denom_b = jnp.sum(onehot_t * denom, axis=0, keepdims=True)
            a = ee / (denom_b + 1e-16)
            r = jnp.dot(xn * a, onehot, preferred_element_type=_f32)
            q_star = jnp.concatenate([q, r], axis=0)
        qs_ref[s] = q_star
        if s >= 1:
            exp_ref[s - 1] = jnp.dot(q_star, onehot_t,
                                     preferred_element_type=_f32)


def _s2s_call(xa_t, xb_t, xc_t, brow, bcol, wih_s, whh_s, bias_s):
    return pl.pallas_call(
        _s2s_body,
        out_shape=(
            jax.ShapeDtypeStruct((3, 2 * _D, _G), _f32),
            jax.ShapeDtypeStruct((2, 2 * _D, _N), _f32),
        ),
    )(xa_t, xb_t, xc_t, brow, bcol, wih_s, whh_s, bias_s)


def kernel(x, edge_index, edge_attr, batch, params):
    p = params
    src = edge_index[0]
    dst = edge_index[1]
    ea = jnp.pad(edge_attr, ((0, _EP - _E), (0, 3)))
    src_r = jnp.pad(src, (0, _EP - _E)).reshape(_NW, _NCH, _CHUNK)
    dst_r = jnp.pad(dst, (0, _EP - _E),
                    constant_values=_N).reshape(_NW, _NCH, _CHUNK)
    zeros16 = jnp.zeros((_NT, _D), _f32)
    zeros64 = jnp.zeros((_NT, 4 * _D), _f32)
    ones_msg = jnp.ones((_NW, _NCH, _CHUNK, _D), _f32)

    cnt2 = _sc_scatter(dst_r, ones_msg, zeros16, _D)

    cw1 = jnp.pad(p['conv_W1'], ((0, 3), (0, 0)))
    cb1 = p['conv_b1'].reshape(1, 128)
    cw2 = p['conv_W2']
    cb2 = p['conv_b2'].reshape(1, _D * _D)
    cbias = p['conv_bias'].reshape(1, _D)
    wih_t = p['gru_Wih'].T
    whh_t = p['gru_Whh'].T
    bih = p['gru_bih'].reshape(1, 3 * _D)
    bhh = p['gru_bhh'].reshape(1, 3 * _D)

    out = _lin0_call(x, p['lin0_W'], p['lin0_b'])
    h = out
    for _ in range(3):
        xs = _sc_gather(src_r, out).reshape(_EP, _D)
        msg = _msg_call(ea, xs, cw1, cb1, cw2, cb2)
        agg2 = _sc_scatter(dst_r, msg.reshape(_NW, _NCH, _CHUNK, _D),
                           zeros16, _D)
        h = _gru_call(agg2, cnt2, cbias, wih_t, whh_t, bih, bhh, h)
        out = h

    heads = ['node_mu', 'node_lv', 'graph_mu', 'graph_lv']
    w1s = jnp.stack([jnp.pad(p[n + '_W1'], ((0, 3), (0, 0))) for n in heads])
    b1s = jnp.stack([p[n + '_b1'].reshape(1, 128) for n in heads])
    w2s = jnp.stack([p[n + '_W2'] for n in heads])
    b2s = jnp.stack([p[n + '_b2'].reshape(1, _D * _D) for n in heads])
    biases4 = jnp.concatenate([p[n + '_bias'] for n in heads]).reshape(1, 4 * _D)

    xs = _sc_gather(src_r, out).reshape(_EP, _D)
    msg4 = _msg4_call(ea, xs, w1s, b1s, w2s, b2s)
    agg4 = _sc_scatter(dst_r, msg4.reshape(_NW, _NCH, _CHUNK, 4 * _D),
                       zeros64, 4 * _D)
    hout = _heads_call(agg4, cnt2, biases4)

    node_mu = hout[:, 0:_D]
    node_lv = hout[:, _D:2 * _D]
    gmu_id = hout[:, 2 * _D:3 * _D]
    glv_id = hout[:, 3 * _D:4 * _D]

    s2s = ['s2s_nodes', 's2s_mu', 's2s_lv']
    wih_s = jnp.stack([p[n + '_Wih'] for n in s2s])
    whh_s = jnp.stack([p[n + '_Whh'] for n in s2s])
    bias_s = jnp.stack([(p[n + '_bih'] + p[n + '_bhh']).reshape(4 * _D, 1)
                        for n in s2s])
    brow = batch.reshape(1, _N)
    bcol = batch.reshape(_N, 1)
    qs, exp = _s2s_call(out.T, gmu_id.T, glv_id.T, brow, bcol,
                        wih_s, whh_s, bias_s)

    node_graph = qs[0].T
    grouped_mu_expanded = exp[0].T
    grouped_lv_expanded = exp[1].T
    return (node_mu, node_lv, grouped_mu_expanded, grouped_lv_expanded,
            node_graph)


# speedup vs baseline: 4.3719x; 1.0045x over previous
import functools

import jax
import jax.numpy as jnp
from jax import lax
from jax.experimental import pallas as pl
from jax.experimental.pallas import tpu as pltpu
from jax.experimental.pallas import tpu_sc as plsc

_N = 10000
_E = 160000
_F = 128
_D = 16
_G = 64

_NCORE = 2
_NSUB = 16
_NW = _NCORE * _NSUB
_CHUNK = 128
_NCH = 40
_EP = _NW * _NCH * _CHUNK
_NT = 10240
_RPS = _NT // _NSUB

_BE = 2048
_GE = _EP // _BE
_BN = 2000
_GN = _N // _BN

_f32 = jnp.float32


def _mesh():
    return plsc.VectorSubcoreMesh(
        core_axis_name="c", subcore_axis_name="s",
        num_cores=_NCORE, num_subcores=_NSUB)


def _sc_gather_body(src_hbm, x_hbm, out_hbm, idx_v, rows_v, sem):
    cid = lax.axis_index("c")
    sid = lax.axis_index("s")
    wid = sid * _NCORE + cid
    pltpu.sync_copy(src_hbm.at[wid], idx_v)

    @pl.loop(0, _NCH + 8, step=8)
    def _grp(j0):
        @pl.when(j0 < _NCH)
        def _fire():
            for b in range(8):
                pltpu.async_copy(x_hbm.at[idx_v.at[j0 + b]],
                                 rows_v.at[j0 + b], sem)

        @pl.when(j0 >= 8)
        def _drain():
            for b in range(8):
                pltpu.make_async_copy(x_hbm.at[idx_v.at[j0 - 8 + b]],
                                      rows_v.at[j0 - 8 + b], sem).wait()

    pltpu.sync_copy(rows_v, out_hbm.at[wid])


def _sc_gather(src_r, x):
    return pl.kernel(
        _sc_gather_body,
        out_type=jax.ShapeDtypeStruct((_NW, _NCH, _CHUNK, _D), _f32),
        mesh=_mesh(),
        compiler_params=pltpu.CompilerParams(use_tc_tiling_on_sc=False),
        scratch_types=[
            pltpu.VMEM((_NCH, _CHUNK), jnp.int32),
            pltpu.VMEM((_NCH, _CHUNK, _D), _f32),
            pltpu.SemaphoreType.DMA,
        ],
    )(src_r, x)


def _sc_scatter_body(dst_hbm, msg_hbm, zeros_hbm, out_hbm, idx_v, msg_v, table,
                     lsem):
    cid = lax.axis_index("c")
    sid = lax.axis_index("s")
    wid = sid * _NCORE + cid
    rows = pl.ds(sid * _RPS, _RPS)
    pltpu.sync_copy(zeros_hbm.at[rows], table.at[rows])
    pltpu.sync_copy(dst_hbm.at[wid], idx_v)
    plsc.subcore_barrier()
    pltpu.async_copy(msg_hbm.at[wid, 0], msg_v.at[0], lsem)

    @pl.loop(0, _NCH)
    def _chunk(j):
        b = lax.rem(j, 2)
        pltpu.make_async_copy(msg_hbm.at[wid, j], msg_v.at[b], lsem).wait()

        @pl.when(j < _NCH - 1)
        def _prefetch():
            pltpu.async_copy(msg_hbm.at[wid, j + 1], msg_v.at[1 - b], lsem)

        pltpu.sync_copy(msg_v.at[b], table.at[idx_v.at[j]], add=True)

    plsc.subcore_barrier()
    pltpu.sync_copy(table.at[rows], out_hbm.at[cid].at[rows])


def _sc_scatter(dst_r, msg_r, zeros, width):
    return pl.kernel(
        _sc_scatter_body,
        out_type=jax.ShapeDtypeStruct((_NCORE, _NT, width), _f32),
        mesh=_mesh(),
        compiler_params=pltpu.CompilerParams(use_tc_tiling_on_sc=False),
        scratch_types=[
            pltpu.VMEM((_NCH, _CHUNK), jnp.int32),
            pltpu.VMEM((2, _CHUNK, width), _f32),
            pltpu.VMEM_SHARED((_NT, width), _f32),
            pltpu.SemaphoreType.DMA,
        ],
    )(dst_r, msg_r, zeros)


def _lin0_body(x_ref, w_ref, b_ref, o_ref):
    y = jnp.dot(x_ref[...], w_ref[...], preferred_element_type=_f32)
    o_ref[...] = jnp.maximum(y + b_ref[...], 0.0)


def _lin0_call(x, w, b):
    return pl.pallas_call(
        _lin0_body,
        grid=(_GN,),
        in_specs=[
            pl.BlockSpec((_BN, _F), lambda i: (i, 0)),
            pl.BlockSpec((_F, _D), lambda i: (0, 0)),
            pl.BlockSpec((1, _D), lambda i: (0, 0)),
        ],
        out_specs=pl.BlockSpec((_BN, _D), lambda i: (i, 0)),
        out_shape=jax.ShapeDtypeStruct((_N, _D), _f32),
    )(x, w, b.reshape(1, _D))


def _spread_fold():
    ri = lax.broadcasted_iota(jnp.int32, (_D, _D * _D), 0)
    rc = lax.broadcasted_iota(jnp.int32, (_D, _D * _D), 1)
    spread = (ri == rc // _D).astype(_f32)
    si = lax.broadcasted_iota(jnp.int32, (_D * _D, _D), 0)
    sc = lax.broadcasted_iota(jnp.int32, (_D * _D, _D), 1)
    fold = (si % _D == sc).astype(_f32)
    return spread, fold


def _edge_msg(ea, xs, w1, b1, w2, b2, spread, fold):
    h = jnp.maximum(jnp.dot(ea, w1, preferred_element_type=_f32) + b1, 0.0)
    w = jnp.dot(h.astype(jnp.bfloat16), w2.astype(jnp.bfloat16),
                preferred_element_type=_f32) + b2
    xr = jnp.dot(xs, spread, preferred_element_type=_f32)
    return jnp.dot(xr * w, fold, preferred_element_type=_f32)


def _msg_body(ea_ref, xs_ref, w1_ref, b1_ref, w2_ref, b2_ref, o_ref):
    spread, fold = _spread_fold()
    o_ref[...] = _edge_msg(ea_ref[...], xs_ref[...], w1_ref[...], b1_ref[...],
                           w2_ref[...], b2_ref[...], spread, fold)


def _msg_call(ea, xs, w1, b1, w2, b2):
    return pl.pallas_call(
        _msg_body,
        grid=(_GE,),
        in_specs=[
            pl.BlockSpec((_BE, 8), lambda i: (i, 0)),
            pl.BlockSpec((_BE, _D), lambda i: (i, 0)),
            pl.BlockSpec((8, 128), lambda i: (0, 0)),
            pl.BlockSpec((1, 128), lambda i: (0, 0)),
            pl.BlockSpec((128, _D * _D), lambda i: (0, 0)),
            pl.BlockSpec((1, _D * _D), lambda i: (0, 0)),
        ],
        out_specs=pl.BlockSpec((_BE, _D), lambda i: (i, 0)),
        out_shape=jax.ShapeDtypeStruct((_EP, _D), _f32),
    )(ea, xs, w1, b1, w2, b2)


def _msg_ones_body(ea_ref, xs_ref, w1_ref, b1_ref, w2_ref, b2_ref, o_ref):
    spread, fold = _spread_fold()
    o_ref[:, 0:_D] = _edge_msg(ea_ref[...], xs_ref[...], w1_ref[...],
                               b1_ref[...], w2_ref[...], b2_ref[...],
                               spread, fold)
    o_ref[:, _D:2 * _D] = jnp.ones((_BE, _D), _f32)


def _msg_ones_call(ea, xs, w1, b1, w2, b2):
    return pl.pallas_call(
        _msg_ones_body,
        grid=(_GE,),
        in_specs=[
            pl.BlockSpec((_BE, 8), lambda i: (i, 0)),
            pl.BlockSpec((_BE, _D), lambda i: (i, 0)),
            pl.BlockSpec((8, 128), lambda i: (0, 0)),
            pl.BlockSpec((1, 128), lambda i: (0, 0)),
            pl.BlockSpec((128, _D * _D), lambda i: (0, 0)),
            pl.BlockSpec((1, _D * _D), lambda i: (0, 0)),
        ],
        out_specs=pl.BlockSpec((_BE, 2 * _D), lambda i: (i, 0)),
        out_shape=jax.ShapeDtypeStruct((_EP, 2 * _D), _f32),
    )(ea, xs, w1, b1, w2, b2)


def _msg4_body(ea_ref, xs_ref, w1_ref, b1_ref, w2_ref, b2_ref, o_ref):
    ea = ea_ref[...]
    xs = xs_ref[...]
    spread, fold = _spread_fold()
    for hd in range(4):
        o_ref[:, hd * _D:(hd + 1) * _D] = _edge_msg(
            ea, xs, w1_ref[hd], b1_ref[hd], w2_ref[hd], b2_ref[hd],
            spread, fold)


def _msg4_call(ea, xs, w1s, b1s, w2s, b2s):
    return pl.pallas_call(
        _msg4_body,
        grid=(_GE,),
        in_specs=[
            pl.BlockSpec((_BE, 8), lambda i: (i, 0)),
            pl.BlockSpec((_BE, _D), lambda i: (i, 0)),
            pl.BlockSpec((4, 8, 128), lambda i: (0, 0, 0)),
            pl.BlockSpec((4, 1, 128), lambda i: (0, 0, 0)),
            pl.BlockSpec((4, 128, _D * _D), lambda i: (0, 0, 0)),
            pl.BlockSpec((4, 1, _D * _D), lambda i: (0, 0, 0)),
        ],
        out_specs=pl.BlockSpec((_BE, 4 * _D), lambda i: (i, 0)),
        out_shape=jax.ShapeDtypeStruct((_EP, 4 * _D), _f32),
    )(ea, xs, w1s, b1s, w2s, b2s)


def _gru_body(aa_ref, ab_ref, ca_ref, cb_ref, bias_ref, wih_ref, whh_ref,
              bih_ref, bhh_ref, h_ref, o_ref):
    cnt = jnp.maximum(ca_ref[...] + cb_ref[...], 1.0)
    m = jnp.maximum((aa_ref[...] + ab_ref[...]) / cnt + bias_ref[...], 0.0)
    h = h_ref[...]
    gi = jnp.dot(m, wih_ref[...], preferred_element_type=_f32) + bih_ref[...]
    gh = jnp.dot(h, whh_ref[...], preferred_element_type=_f32) + bhh_ref[...]
    r = jax.nn.sigmoid(gi[:, 0:_D] + gh[:, 0:_D])
    z = jax.nn.sigmoid(gi[:, _D:2 * _D] + gh[:, _D:2 * _D])
    n = jnp.tanh(gi[:, 2 * _D:3 * _D] + r * gh[:, 2 * _D:3 * _D])
    o_ref[...] = (1.0 - z) * n + z * h


def _gru_call(agg2, cnt2, bias, wih_t, whh_t, bih, bhh, h):
    return pl.pallas_call(
        _gru_body,
        grid=(_GN,),
        in_specs=[
            pl.BlockSpec((_BN, _D), lambda i: (i, 0)),
            pl.BlockSpec((_BN, _D), lambda i: (i, 0)),
            pl.BlockSpec((_BN, _D), lambda i: (i, 0)),
            pl.BlockSpec((_BN, _D), lambda i: (i, 0)),
            pl.BlockSpec((1, _D), lambda i: (0, 0)),
            pl.BlockSpec((_D, 3 * _D), lambda i: (0, 0)),
            pl.BlockSpec((_D, 3 * _D), lambda i: (0, 0)),
            pl.BlockSpec((1, 3 * _D), lambda i: (0, 0)),
            pl.BlockSpec((1, 3 * _D), lambda i: (0, 0)),
            pl.BlockSpec((_BN, _D), lambda i: (i, 0)),
        ],
        out_specs=pl.BlockSpec((_BN, _D), lambda i: (i, 0)),
        out_shape=jax.ShapeDtypeStruct((_N, _D), _f32),
    )(agg2[0], agg2[1], cnt2[0], cnt2[1], bias, wih_t, whh_t, bih, bhh, h)


def _heads_body(aa_ref, ab_ref, ca_ref, cb_ref, bias_ref, o_ref):
    cnt = jnp.maximum(ca_ref[...] + cb_ref[...], 1.0)
    cnt4 = jnp.concatenate([cnt, cnt, cnt, cnt], axis=1)
    o_ref[...] = jnp.maximum(
        (aa_ref[...] + ab_ref[...]) / cnt4 + bias_ref[...], 0.0)


def _heads_call(agg4, cnt2, biases):
    return pl.pallas_call(
        _heads_body,
        grid=(_GN,),
        in_specs=[
            pl.BlockSpec((_BN, 4 * _D), lambda i: (i, 0)),
            pl.BlockSpec((_BN, 4 * _D), lambda i: (i, 0)),
            pl.BlockSpec((_BN, _D), lambda i: (i, 0)),
            pl.BlockSpec((_BN, _D), lambda i: (i, 0)),
            pl.BlockSpec((1, 4 * _D), lambda i: (0, 0)),
        ],
        out_specs=pl.BlockSpec((_BN, 4 * _D), lambda i: (i, 0)),
        out_shape=jax.ShapeDtypeStruct((_N, 4 * _D), _f32),
    )(agg4[0], agg4[1], cnt2[0], cnt2[1], biases)


def _s2s_body(xa_ref, xb_ref, xc_ref, brow_ref, bcol_ref, wih_ref, whh_ref,
              bias_ref, qs_ref, exp_ref):
    iota_gc = lax.broadcasted_iota(jnp.int32, (_G, 1), 0)
    iota_gr = lax.broadcasted_iota(jnp.int32, (1, _G), 1)
    mask_t = brow_ref[...] == iota_gc
    onehot_t = mask_t.astype(_f32)
    onehot = (bcol_ref[...] == iota_gr).astype(_f32)
    xns = (xa_ref, xb_ref, xc_ref)
    for s in range(3):
        xn = xns[s][...]
        wih = wih_ref[s]
        whh = whh_ref[s]
        bias = bias_ref[s]
        q_star = jnp.zeros((2 * _D, _G), _f32)
        h = jnp.zeros((_D, _G), _f32)
        c = jnp.zeros((_D, _G), _f32)
        for _step in range(3):
            gates = (jnp.dot(wih, q_star, preferred_element_type=_f32)
                     + jnp.dot(whh, h, preferred_element_type=_f32) + bias)
            gi = jax.nn.sigmoid(gates[0:_D])
            gf = jax.nn.sigmoid(gates[_D:2 * _D])
            gg = jnp.tanh(gates[2 * _D:3 * _D])
            go = jax.nn.sigmoid(gates[3 * _D:4 * _D])
            c = gf * c + gi * gg
            h = go * jnp.tanh(c)
            q = h
            qb = jnp.dot(q, onehot_t, preferred_element_type=_f32)
            e = jnp.sum(xn * qb, axis=0, keepdims=True)
            masked = jnp.where(mask_t, jnp.broadcast_to(e, (_G, _N)), -1e30)
            emax = jnp.max(masked, axis=1, keepdims=True)
            emax = jnp.where(emax > -1e29, emax, 0.0)
            emax_b = jnp.sum(onehot_t * emax, axis=0, keepdims=True)
            ee = jnp.exp(e - emax_b)
            denom = jnp.sum(onehot_t * ee, axis=1, keepdims=True)
            denom_b = jnp.sum(onehot_t * denom, axis=0, keepdims=True)
            a = ee / (denom_b + 1e-16)
            r = jnp.dot(xn * a, onehot, preferred_element_type=_f32)
            q_star = jnp.concatenate([q, r], axis=0)
        qs_ref[s] = q_star
        if s >= 1:
            exp_ref[s - 1] = jnp.dot(q_star, onehot_t,
                                     preferred_element_type=_f32)


def _s2s_call(xa_t, xb_t, xc_t, brow, bcol, wih_s, whh_s, bias_s):
    return pl.pallas_call(
        _s2s_body,
        out_shape=(
            jax.ShapeDtypeStruct((3, 2 * _D, _G), _f32),
            jax.ShapeDtypeStruct((2, 2 * _D, _N), _f32),
        ),
    )(xa_t, xb_t, xc_t, brow, bcol, wih_s, whh_s, bias_s)


def kernel(x, edge_index, edge_attr, batch, params):
    p = params
    src = edge_index[0]
    dst = edge_index[1]
    ea = jnp.pad(edge_attr, ((0, _EP - _E), (0, 3)))
    src_r = jnp.pad(src, (0, _EP - _E)).reshape(_NW, _NCH, _CHUNK)
    dst_r = jnp.pad(dst, (0, _EP - _E),
                    constant_values=_N).reshape(_NW, _NCH, _CHUNK)
    zeros16 = jnp.zeros((_NT, _D), _f32)
    zeros32 = jnp.zeros((_NT, 2 * _D), _f32)
    zeros64 = jnp.zeros((_NT, 4 * _D), _f32)

    cw1 = jnp.pad(p['conv_W1'], ((0, 3), (0, 0)))
    cb1 = p['conv_b1'].reshape(1, 128)
    cw2 = p['conv_W2']
    cb2 = p['conv_b2'].reshape(1, _D * _D)
    cbias = p['conv_bias'].reshape(1, _D)
    wih_t = p['gru_Wih'].T
    whh_t = p['gru_Whh'].T
    bih = p['gru_bih'].reshape(1, 3 * _D)
    bhh = p['gru_bhh'].reshape(1, 3 * _D)

    out = _lin0_call(x, p['lin0_W'], p['lin0_b'])
    h = out

    xs = _sc_gather(src_r, out).reshape(_EP, _D)
    msgw = _msg_ones_call(ea, xs, cw1, cb1, cw2, cb2)
    aggw = _sc_scatter(dst_r, msgw.reshape(_NW, _NCH, _CHUNK, 2 * _D),
                       zeros32, 2 * _D)
    agg2 = aggw[:, :, 0:_D]
    cnt2 = aggw[:, :, _D:2 * _D]
    h = _gru_call(agg2, cnt2, cbias, wih_t, whh_t, bih, bhh, h)
    out = h

    for _ in range(2):
        xs = _sc_gather(src_r, out).reshape(_EP, _D)
        msg = _msg_call(ea, xs, cw1, cb1, cw2, cb2)
        agg2 = _sc_scatter(dst_r, msg.reshape(_NW, _NCH, _CHUNK, _D),
                           zeros16, _D)
        h = _gru_call(agg2, cnt2, cbias, wih_t, whh_t, bih, bhh, h)
        out = h

    heads = ['node_mu', 'node_lv', 'graph_mu', 'graph_lv']
    w1s = jnp.stack([jnp.pad(p[n + '_W1'], ((0, 3), (0, 0))) for n in heads])
    b1s = jnp.stack([p[n + '_b1'].reshape(1, 128) for n in heads])
    w2s = jnp.stack([p[n + '_W2'] for n in heads])
    b2s = jnp.stack([p[n + '_b2'].reshape(1, _D * _D) for n in heads])
    biases4 = jnp.concatenate([p[n + '_bias'] for n in heads]).reshape(1, 4 * _D)

    xs = _sc_gather(src_r, out).reshape(_EP, _D)
    msg4 = _msg4_call(ea, xs, w1s, b1s, w2s, b2s)
    agg4 = _sc_scatter(dst_r, msg4.reshape(_NW, _NCH, _CHUNK, 4 * _D),
                       zeros64, 4 * _D)
    hout = _heads_call(agg4, cnt2, biases4)

    node_mu = hout[:, 0:_D]
    node_lv = hout[:, _D:2 * _D]
    gmu_id = hout[:, 2 * _D:3 * _D]
    glv_id = hout[:, 3 * _D:4 * _D]

    s2s = ['s2s_nodes', 's2s_mu', 's2s_lv']
    wih_s = jnp.stack([p[n + '_Wih'] for n in s2s])
    whh_s = jnp.stack([p[n + '_Whh'] for n in s2s])
    bias_s = jnp.stack([(p[n + '_bih'] + p[n + '_bhh']).reshape(4 * _D, 1)
                        for n in s2s])
    brow = batch.reshape(1, _N)
    bcol = batch.reshape(_N, 1)
    qs, exp = _s2s_call(out.T, gmu_id.T, glv_id.T, brow, bcol,
                        wih_s, whh_s, bias_s)

    node_graph = qs[0].T
    grouped_mu_expanded = exp[0].T
    grouped_lv_expanded = exp[1].T
    return (node_mu, node_lv, grouped_mu_expanded, grouped_lv_expanded,
            node_graph)
